# baseline (speedup 1.0000x reference)
import functools

import jax
import jax.numpy as jnp
from jax import lax
from jax.experimental import pallas as pl
from jax.experimental.pallas import tpu as pltpu
from jax.experimental.pallas import tpu_sc as plsc

_HIDDEN = 8192
_LANES = 16
_NW = 32
_G = 2


def _combo_table(type_table, juris_table):
    n_types, hidden = type_table.shape
    n_juris = juris_table.shape[0]

    def body(t_ref, j_ref, o_ref):
        o_ref[...] = t_ref[...] + j_ref[...]

    out = pl.pallas_call(
        body,
        grid=(n_types * n_juris,),
        in_specs=[
            pl.BlockSpec((1, 1, hidden), lambda r: (r // n_juris, 0, 0)),
            pl.BlockSpec((1, 1, hidden), lambda r: (r % n_juris, 0, 0)),
        ],
        out_specs=pl.BlockSpec((1, 1, hidden), lambda r: (r, 0, 0)),
        out_shape=jax.ShapeDtypeStruct((n_types * n_juris, 1, hidden),
                                       jnp.float32),
    )(type_table.reshape(n_types, 1, hidden),
      juris_table.reshape(n_juris, 1, hidden))
    return out.reshape(n_types * n_juris, hidden)


_NBUF = 2
_UNROLL = 4


def _make_sc_lookup(n_b, n_l):
    n_tokens = n_b * n_l
    wb = n_b // _NW
    bgroups = wb // _G
    groups = n_tokens // (_NW * _G)
    mesh = plsc.VectorSubcoreMesh(core_axis_name="c", subcore_axis_name="s")
    nc = mesh.num_cores

    @functools.partial(
        pl.kernel,
        mesh=mesh,
        out_type=jax.ShapeDtypeStruct((n_l, n_b, _HIDDEN), jnp.float32),
        scratch_types=[
            pltpu.VMEM((groups, 16), jnp.int32),
            [pltpu.VMEM((_G, _HIDDEN), jnp.float32)] * _NBUF,
            [pltpu.VMEM((_G, _HIDDEN // 2), jnp.int32)] * _NBUF,
            [pltpu.VMEM((_G, _HIDDEN), jnp.float32)] * _NBUF,
            [pltpu.SemaphoreType.DMA] * _NBUF,
            [pltpu.SemaphoreType.DMA] * _NBUF,
            [pltpu.SemaphoreType.DMA] * _NBUF,
        ],
    )
    def lookup(ent_hbm, combo_hbm, ids_hbm, out_hbm,
               idx_v, ebufs, cbufs, obufs, esems, csems, osems):
        wid = lax.axis_index("s") * nc + lax.axis_index("c")
        base_b = wid * wb
        pltpu.sync_copy(ids_hbm.at[wid], idx_v)

        def out_dst(g):
            li = g // bgroups
            bi = base_b + (g - li * bgroups) * _G
            return out_hbm.at[li, pl.ds(bi, _G)]

        def gather_pair(g, b):
            pltpu.async_copy(
                ent_hbm.at[idx_v.at[g, pl.ds(0, _G)]], ebufs[b], esems[b])
            pltpu.async_copy(
                combo_hbm.at[idx_v.at[g, pl.ds(8, _G)]], cbufs[b], csems[b])

        def wait_pair(g, b):
            pltpu.make_async_copy(
                ent_hbm.at[idx_v.at[g, pl.ds(0, _G)]], ebufs[b],
                esems[b]).wait()
            pltpu.make_async_copy(
                combo_hbm.at[idx_v.at[g, pl.ds(8, _G)]], cbufs[b],
                csems[b]).wait()

        for b in range(_NBUF):
            gather_pair(b, b)

        def step(it, carry):
            for b in range(_NBUF):
                g = it * _NBUF + b
                wait_pair(g, b)

                @pl.when(it >= 1)
                def _drain(g=g, b=b):
                    pltpu.make_async_copy(
                        obufs[b], out_dst(g - _NBUF), osems[b]).wait()

                eb, cb, ob = ebufs[b], cbufs[b], obufs[b]
                for r in range(_G):
                    def add_body(i, c2, r=r, eb=eb, cb=cb, ob=ob):
                        for u in range(_UNROLL):
                            offh = pl.multiple_of(
                                (i * _UNROLL + u) * _LANES, _LANES)
                            off = pl.multiple_of(offh * 2, 2 * _LANES)
                            cv = cb[r, pl.ds(offh, _LANES)]
                            a0 = lax.bitcast_convert_type(
                                lax.shift_left(cv, 16), jnp.float32)
                            a1 = lax.bitcast_convert_type(
                                lax.bitwise_and(cv, jnp.int32(-65536)),
                                jnp.float32)
                            ob[r, pl.ds(off, _LANES)] = (
                                eb[r, pl.ds(off, _LANES)] + a0)
                            ob[r, pl.ds(off + _LANES, _LANES)] = (
                                eb[r, pl.ds(off + _LANES, _LANES)] + a1)
                        return c2
                    lax.fori_loop(0, _HIDDEN // (_UNROLL * 2 * _LANES),
                                  add_body, 0)

                @pl.when(g + _NBUF < groups)
                def _issue(g=g, b=b):
                    gather_pair(g + _NBUF, b)
                pltpu.async_copy(ob, out_dst(g), osems[b])
            return carry

        lax.fori_loop(0, groups // _NBUF, step, 0)
        for b in range(_NBUF):
            pltpu.make_async_copy(
                obufs[b], out_dst(groups - _NBUF + b), osems[b]).wait()

    return lookup


def kernel(entity_ids, entity_types, jurisdictions, entity_table,
           type_table, juris_table):
    b, l = entity_ids.shape
    n_tokens = b * l
    n_juris = juris_table.shape[0]
    wb = b // _NW
    groups = n_tokens // (_NW * _G)

    def order(a):
        return (a.astype(jnp.int32).reshape(_NW, wb, l)
                .transpose(0, 2, 1).reshape(_NW, groups, _G))

    eids = order(entity_ids)
    cids = order(entity_types * n_juris + jurisdictions)
    zpad = jnp.zeros((_NW, groups, 8 - _G), jnp.int32)
    ids = jnp.concatenate([eids, zpad, cids, zpad], axis=2)

    combo = _combo_table(type_table, juris_table)
    combo_bf = (combo.reshape(-1, _HIDDEN // (2 * _LANES), 2, _LANES)
                .transpose(0, 1, 3, 2)
                .reshape(-1, _HIDDEN // 2, 2).astype(jnp.bfloat16))
    combo_i32 = lax.bitcast_convert_type(combo_bf, jnp.int32)
    out_lm = _make_sc_lookup(b, l)(entity_table, combo_i32, ids)
    return jnp.transpose(out_lm, (1, 0, 2))


# speedup vs baseline: 2.1337x; 2.1337x over previous
import functools

import jax
import jax.numpy as jnp
from jax import lax
from jax.experimental import pallas as pl
from jax.experimental.pallas import tpu as pltpu
from jax.experimental.pallas import tpu_sc as plsc

_HIDDEN = 8192
_LANES = 16
_NW = 32
_G = 2


def _combo_table(type_table, juris_table):
    n_types, hidden = type_table.shape
    n_juris = juris_table.shape[0]

    def body(t_ref, j_ref, o_ref):
        o_ref[...] = t_ref[...] + j_ref[...]

    out = pl.pallas_call(
        body,
        grid=(n_types * n_juris,),
        in_specs=[
            pl.BlockSpec((1, 1, hidden), lambda r: (r // n_juris, 0, 0)),
            pl.BlockSpec((1, 1, hidden), lambda r: (r % n_juris, 0, 0)),
        ],
        out_specs=pl.BlockSpec((1, 1, hidden), lambda r: (r, 0, 0)),
        out_shape=jax.ShapeDtypeStruct((n_types * n_juris, 1, hidden),
                                       jnp.float32),
    )(type_table.reshape(n_types, 1, hidden),
      juris_table.reshape(n_juris, 1, hidden))
    return out.reshape(n_types * n_juris, hidden)


_NBUF = 2
_UNROLL = 8


def _make_sc_lookup(n_b, n_l):
    n_tokens = n_b * n_l
    wb = n_b // _NW
    bgroups = wb // _G
    groups = n_tokens // (_NW * _G)
    mesh = plsc.VectorSubcoreMesh(core_axis_name="c", subcore_axis_name="s")
    nc = mesh.num_cores

    @functools.partial(
        pl.kernel,
        mesh=mesh,
        out_type=jax.ShapeDtypeStruct((n_l, n_b, _HIDDEN), jnp.float32),
        scratch_types=[
            pltpu.VMEM((groups // 2, 32), jnp.int32),
            [pltpu.VMEM((_G, _HIDDEN), jnp.float32)] * _NBUF,
            [pltpu.VMEM((_G, _HIDDEN), jnp.float32)] * _NBUF,
            [pltpu.VMEM((_G, _HIDDEN), jnp.float32)] * _NBUF,
            [pltpu.SemaphoreType.DMA] * _NBUF,
            [pltpu.SemaphoreType.DMA] * _NBUF,
            [pltpu.SemaphoreType.DMA] * _NBUF,
        ],
    )
    def lookup(ent_hbm, combo_hbm, ids_hbm, out_hbm,
               idx_v, ebufs, cbufs, obufs, esems, csems, osems):
        wid = lax.axis_index("s") * nc + lax.axis_index("c")
        base_b = wid * wb
        pltpu.sync_copy(ids_hbm.at[wid], idx_v)

        def out_dst(g):
            li = g // bgroups
            bi = base_b + (g - li * bgroups) * _G
            return out_hbm.at[li, pl.ds(bi, _G)]

        def eslice(g):
            off = pl.multiple_of((g % 2) * 16, 8)
            return idx_v.at[g // 2, pl.ds(off, _G)]

        def cslice(g):
            off = pl.multiple_of((g % 2) * 16 + 8, 8)
            return idx_v.at[g // 2, pl.ds(off, _G)]

        def gather_pair(g, b):
            pltpu.async_copy(ent_hbm.at[eslice(g)], ebufs[b], esems[b])
            pltpu.async_copy(combo_hbm.at[cslice(g)], cbufs[b], csems[b])

        def wait_pair(g, b):
            pltpu.make_async_copy(
                ent_hbm.at[eslice(g)], ebufs[b], esems[b]).wait()
            pltpu.make_async_copy(
                combo_hbm.at[cslice(g)], cbufs[b], csems[b]).wait()

        for b in range(_NBUF):
            gather_pair(b, b)

        def step(it, carry):
            for b in range(_NBUF):
                g = it * _NBUF + b
                wait_pair(g, b)

                @pl.when(it >= 1)
                def _drain(g=g, b=b):
                    pltpu.make_async_copy(
                        obufs[b], out_dst(g - _NBUF), osems[b]).wait()

                eb, cb, ob = ebufs[b], cbufs[b], obufs[b]
                for r in range(_G):
                    def add_body(i, c2, r=r, eb=eb, cb=cb, ob=ob):
                        for u in range(_UNROLL):
                            off = i * (_UNROLL * _LANES) + u * _LANES
                            ob[r, pl.ds(off, _LANES)] = (
                                eb[r, pl.ds(off, _LANES)]
                                + cb[r, pl.ds(off, _LANES)])
                        return c2
                    lax.fori_loop(0, _HIDDEN // (_UNROLL * _LANES),
                                  add_body, 0)

                @pl.when(g + _NBUF < groups)
                def _issue(g=g, b=b):
                    gather_pair(g + _NBUF, b)
                pltpu.async_copy(ob, out_dst(g), osems[b])
            return carry

        lax.fori_loop(0, groups // _NBUF, step, 0)
        for b in range(_NBUF):
            pltpu.make_async_copy(
                obufs[b], out_dst(groups - _NBUF + b), osems[b]).wait()

    return lookup


def kernel(entity_ids, entity_types, jurisdictions, entity_table,
           type_table, juris_table):
    b, l = entity_ids.shape
    n_tokens = b * l
    n_juris = juris_table.shape[0]
    wb = b // _NW
    groups = n_tokens // (_NW * _G)

    def order(a):
        return (a.astype(jnp.int32).reshape(_NW, wb, l)
                .transpose(0, 2, 1).reshape(_NW, groups, _G))

    eids = order(entity_ids)
    cids = order(entity_types * n_juris + jurisdictions)
    zpad = jnp.zeros((_NW, groups, 8 - _G), jnp.int32)
    ids = (jnp.concatenate([eids, zpad, cids, zpad], axis=2)
           .reshape(_NW, groups // 2, 32))

    combo = _combo_table(type_table, juris_table)
    out_lm = _make_sc_lookup(b, l)(entity_table, combo, ids)
    return jnp.transpose(out_lm, (1, 0, 2))


# speedup vs baseline: 2.2646x; 1.0613x over previous
import functools

import jax
import jax.numpy as jnp
from jax import lax
from jax.experimental import pallas as pl
from jax.experimental.pallas import tpu as pltpu
from jax.experimental.pallas import tpu_sc as plsc

_HIDDEN = 8192
_LANES = 16
_NW = 32
_G = 2


def _combo_table(type_table, juris_table):
    n_types, hidden = type_table.shape
    n_juris = juris_table.shape[0]

    def body(t_ref, j_ref, o_ref):
        o_ref[...] = t_ref[...][None] + j_ref[...]

    out = pl.pallas_call(
        body,
        grid=(n_juris,),
        in_specs=[
            pl.BlockSpec((n_types, hidden), lambda j: (0, 0)),
            pl.BlockSpec((1, 1, hidden), lambda j: (j, 0, 0)),
        ],
        out_specs=pl.BlockSpec((1, n_types, hidden), lambda j: (j, 0, 0)),
        out_shape=jax.ShapeDtypeStruct((n_juris, n_types, hidden),
                                       jnp.float32),
    )(type_table, juris_table.reshape(n_juris, 1, hidden))
    return out.reshape(n_juris * n_types, hidden)


_NBUF = 2
_UNROLL = 8


def _make_sc_lookup(n_b, n_l):
    n_tokens = n_b * n_l
    wb = n_b // _NW
    bgroups = wb // _G
    groups = n_tokens // (_NW * _G)
    mesh = plsc.VectorSubcoreMesh(core_axis_name="c", subcore_axis_name="s")
    nc = mesh.num_cores

    @functools.partial(
        pl.kernel,
        mesh=mesh,
        out_type=jax.ShapeDtypeStruct((n_l, n_b, _HIDDEN), jnp.float32),
        scratch_types=[
            pltpu.VMEM((groups // 2, 32), jnp.int32),
            [pltpu.VMEM((_G, _HIDDEN), jnp.float32)] * _NBUF,
            [pltpu.VMEM((_G, _HIDDEN), jnp.float32)] * _NBUF,
            [pltpu.VMEM((_G, _HIDDEN), jnp.float32)] * _NBUF,
            [pltpu.SemaphoreType.DMA] * _NBUF,
            [pltpu.SemaphoreType.DMA] * _NBUF,
            [pltpu.SemaphoreType.DMA] * _NBUF,
        ],
    )
    def lookup(ent_hbm, combo_hbm, ids_hbm, out_hbm,
               idx_v, ebufs, cbufs, obufs, esems, csems, osems):
        wid = lax.axis_index("s") * nc + lax.axis_index("c")
        base_b = wid * wb
        pltpu.sync_copy(ids_hbm.at[wid], idx_v)

        def out_dst(g):
            li = g // bgroups
            bi = base_b + (g - li * bgroups) * _G
            return out_hbm.at[li, pl.ds(bi, _G)]

        def eslice(g):
            off = pl.multiple_of((g % 2) * 16, 8)
            return idx_v.at[g // 2, pl.ds(off, _G)]

        def cslice(g):
            off = pl.multiple_of((g % 2) * 16 + 8, 8)
            return idx_v.at[g // 2, pl.ds(off, _G)]

        def gather_pair(g, b):
            pltpu.async_copy(ent_hbm.at[eslice(g)], ebufs[b], esems[b])
            pltpu.async_copy(combo_hbm.at[cslice(g)], cbufs[b], csems[b])

        def wait_pair(g, b):
            pltpu.make_async_copy(
                ent_hbm.at[eslice(g)], ebufs[b], esems[b]).wait()
            pltpu.make_async_copy(
                combo_hbm.at[cslice(g)], cbufs[b], csems[b]).wait()

        for b in range(_NBUF):
            gather_pair(b, b)

        def step(it, carry):
            for b in range(_NBUF):
                g = it * _NBUF + b
                wait_pair(g, b)

                @pl.when(it >= 1)
                def _drain(g=g, b=b):
                    pltpu.make_async_copy(
                        obufs[b], out_dst(g - _NBUF), osems[b]).wait()

                eb, cb, ob = ebufs[b], cbufs[b], obufs[b]
                for r in range(_G):
                    def add_body(i, c2, r=r, eb=eb, cb=cb, ob=ob):
                        for u in range(_UNROLL):
                            off = i * (_UNROLL * _LANES) + u * _LANES
                            ob[r, pl.ds(off, _LANES)] = (
                                eb[r, pl.ds(off, _LANES)]
                                + cb[r, pl.ds(off, _LANES)])
                        return c2
                    lax.fori_loop(0, _HIDDEN // (_UNROLL * _LANES),
                                  add_body, 0)

                @pl.when(g + _NBUF < groups)
                def _issue(g=g, b=b):
                    gather_pair(g + _NBUF, b)
                pltpu.async_copy(ob, out_dst(g), osems[b])
            return carry

        lax.fori_loop(0, groups // _NBUF, step, 0)
        for b in range(_NBUF):
            pltpu.make_async_copy(
                obufs[b], out_dst(groups - _NBUF + b), osems[b]).wait()

    return lookup


def kernel(entity_ids, entity_types, jurisdictions, entity_table,
           type_table, juris_table):
    b, l = entity_ids.shape
    n_tokens = b * l
    n_juris = juris_table.shape[0]
    wb = b // _NW
    groups = n_tokens // (_NW * _G)

    def order(a):
        return (a.astype(jnp.int32).reshape(_NW, wb, l)
                .transpose(0, 2, 1).reshape(_NW, groups, _G))

    eids = order(entity_ids)
    cids = order(jurisdictions * type_table.shape[0] + entity_types)
    zpad = jnp.zeros((_NW, groups, 8 - _G), jnp.int32)
    ids = (jnp.concatenate([eids, zpad, cids, zpad], axis=2)
           .reshape(_NW, groups // 2, 32))

    combo = _combo_table(type_table, juris_table)
    out_lm = _make_sc_lookup(b, l)(entity_table, combo, ids)
    return jnp.transpose(out_lm, (1, 0, 2))
